# baseline (speedup 1.0000x reference)
import jax
import jax.numpy as jnp
from jax import lax
from jax.experimental import pallas as pl
from jax.experimental.pallas import tpu as pltpu
from jax.experimental.pallas import tpu_sc as plsc

N = 320000
D = 128
S = 10000
NW = 32
SPW = 320
S_PAD = NW * SPW
C = 96
SUB = 15
BCH = SUB * C
RPW = SPW + 8
CNT_PAD = ((SPW + 15) // 16 + 1) * 16
NB = 3
RMOD = 4


def _pool_kernel(x_hbm, b_hbm, bounds_hbm, out_hbm,
                 xbuf, bbuf, idxb, acc2, cnt, bnds, acc_sh,
                 sx0, sx1, sx2, ss0, ss1, ss2):
    cid = lax.axis_index("c")
    sid = lax.axis_index("s")
    w = sid * 2 + cid
    seg_lo = w * SPW
    base = sid * RPW

    sx = (sx0, sx1, sx2)
    ss = (ss0, ss1, ss2)

    pltpu.sync_copy(bounds_hbm, bnds)
    bv0 = bnds[pl.ds(w, 16)]
    lo = bv0[0]
    hi = bv0[1]

    zeros16 = jnp.zeros((16,), jnp.float32)

    def zrow(i, carry):
        for j in range(8):
            xbuf[0, i, pl.ds(j * 16, 16)] = zeros16
        return carry
    lax.fori_loop(0, C, zrow, 0)

    def zrow2(i, carry):
        for j in range(8):
            acc2[i, pl.ds(j * 16, 16)] = zeros16
        return carry
    lax.fori_loop(0, RPW, zrow2, 0)
    off0 = 0
    rem_rows = RPW
    while rem_rows > 0:
        sz = min(C, rem_rows)
        pltpu.sync_copy(xbuf.at[0].at[pl.ds(0, sz)],
                        acc_sh.at[pl.ds(base + off0, sz)])
        off0 += sz
        rem_rows -= sz
    for j in range(CNT_PAD // 16):
        cnt[pl.ds(j * 16, 16)] = zeros16

    lo_al = lo & jnp.int32(~7)
    nchunks = (hi - lo_al + C - 1) // C

    def xload(k, b):
        r_eff = pl.multiple_of(jnp.minimum(lo_al + k * C, N - C), 8)
        pltpu.async_copy(x_hbm.at[pl.ds(r_eff, C)], xbuf.at[b], sx[b])

    def xwait(b):
        pltpu.make_async_copy(x_hbm.at[pl.ds(0, C)], xbuf.at[b],
                              sx[b]).wait()

    def scat_wait(b):
        pltpu.make_async_copy(xbuf.at[b], acc_sh.at[idxb.at[b]],
                              ss[b]).wait()

    @pl.when(nchunks > 0)
    def _():
        xload(0, 0)

    def is_valu(k):
        return lax.rem(k, RMOD) == RMOD - 1

    def pair(p, carry):
        for b in range(NB):
            k = NB * p + b

            @pl.when(k < nchunks)
            def _():
                s_sup = k // SUB
                rb_eff = pl.multiple_of(
                    jnp.minimum(lo_al + s_sup * BCH, N - BCH), 8)

                if b == 0:
                    @pl.when(lax.rem(k, SUB) == 0)
                    def _():
                        pltpu.sync_copy(b_hbm.at[pl.ds(rb_eff, BCH)], bbuf)

                @pl.when((k >= 2) & jnp.logical_not(is_valu(k - 2)))
                def _():
                    scat_wait((b + 1) % NB)

                @pl.when(k + 1 < nchunks)
                def _():
                    xload(k + 1, (b + 1) % NB)

                r = lo_al + k * C
                r_eff = pl.multiple_of(jnp.minimum(r, N - C), 8)
                off = r_eff - rb_eff
                vlo = jnp.maximum(r, lo)

                def locvec(j):
                    bvv = bbuf[pl.ds(off + j * 16, 16)]
                    g = r_eff + j * 16 + lax.iota(jnp.int32, 16)
                    valid = (g >= vlo) & (g < hi)
                    loc = jnp.where(valid, bvv - seg_lo, SPW)
                    ones = jnp.where(valid, 1.0, 0.0).astype(jnp.float32)
                    plsc.addupdate_scatter(cnt, [loc], ones)
                    return loc

                @pl.when(jnp.logical_not(is_valu(k)))
                def _():
                    for j in range(C // 16):
                        idxb[b, pl.ds(j * 16, 16)] = base + locvec(j)
                    xwait(b)
                    pltpu.async_copy(xbuf.at[b], acc_sh.at[idxb.at[b]],
                                     ss[b], add=True)

                @pl.when(is_valu(k))
                def _():
                    xwait(b)

                    def group(j, carry2):
                        curloc = carry2[0]
                        s8 = list(carry2[1:])
                        loc = locvec(j)
                        for i in range(16):
                            loc_i = loc[i]
                            pred = loc_i != curloc

                            @pl.when(pred)
                            def _(curloc=curloc, s8=tuple(s8)):
                                for col in range(8):
                                    plsc.addupdate(
                                        acc2.at[curloc,
                                                pl.ds(col * 16, 16)],
                                        s8[col])
                            for col in range(8):
                                xv = xbuf[b, j * 16 + i,
                                          pl.ds(col * 16, 16)]
                                s8[col] = jnp.where(pred, xv,
                                                    s8[col] + xv)
                            curloc = loc_i
                        return (curloc, *s8)

                    fin = lax.fori_loop(
                        0, C // 16, group,
                        (jnp.int32(SPW), *([zeros16] * 8)))
                    for col in range(8):
                        plsc.addupdate(
                            acc2.at[fin[0], pl.ds(col * 16, 16)],
                            fin[1 + col])
        return carry
    lax.fori_loop(0, (nchunks + NB - 1) // NB, pair, 0)

    for b in range(NB):
        cond = jnp.bool_(False)
        for t in range(1, NB):
            cond = cond | ((nchunks >= t)
                           & jnp.logical_not(is_valu(nchunks - t))
                           & (lax.rem(nchunks - t, NB) == b))

        @pl.when(cond)
        def _(b=b):
            scat_wait(b)

    W = 64
    blk = xbuf.at[0].at[pl.ds(0, W)]
    for t in range(SPW // W):
        pltpu.sync_copy(acc_sh.at[pl.ds(base + t * W, W)], blk)

        def div_row(s, carry, t=t):
            cv = cnt[pl.ds(t * W + s, 16)]
            inv = (jnp.ones((16,), jnp.float32) / jnp.maximum(cv, 1.0))[0]
            for j in range(8):
                xbuf[0, s, pl.ds(j * 16, 16)] = (
                    (xbuf[0, s, pl.ds(j * 16, 16)]
                     + acc2[t * W + s, pl.ds(j * 16, 16)]) * inv)
            return carry
        lax.fori_loop(0, W, div_row, 0)
        pltpu.sync_copy(blk, out_hbm.at[pl.ds(seg_lo + t * W, W)])


def kernel(x, batch):
    b32 = batch.astype(jnp.int32)
    edges = jnp.arange(48, dtype=jnp.int32) * SPW
    bounds = jnp.sum((b32[:, None] < edges[None, :]).astype(jnp.int32),
                     axis=0, dtype=jnp.int32)

    mesh = plsc.VectorSubcoreMesh(core_axis_name="c", subcore_axis_name="s")
    out = pl.kernel(
        _pool_kernel,
        mesh=mesh,
        compiler_params=pltpu.CompilerParams(needs_layout_passes=False),
        out_type=jax.ShapeDtypeStruct((S_PAD, D), jnp.float32),
        scratch_types=[
            pltpu.VMEM((NB, C, D), jnp.float32),
            pltpu.VMEM((BCH,), jnp.int32),
            pltpu.VMEM((NB, C), jnp.int32),
            pltpu.VMEM((RPW, D), jnp.float32),
            pltpu.VMEM((CNT_PAD,), jnp.float32),
            pltpu.VMEM((48,), jnp.int32),
            pltpu.VMEM_SHARED((16 * RPW, D), jnp.float32),
            pltpu.SemaphoreType.DMA,
            pltpu.SemaphoreType.DMA,
            pltpu.SemaphoreType.DMA,
            pltpu.SemaphoreType.DMA,
            pltpu.SemaphoreType.DMA,
            pltpu.SemaphoreType.DMA,
        ],
    )(x, b32, bounds)
    return out[:S]


# speedup vs baseline: 1.0897x; 1.0897x over previous
import jax
import jax.numpy as jnp
from jax import lax
from jax.experimental import pallas as pl
from jax.experimental.pallas import tpu as pltpu
from jax.experimental.pallas import tpu_sc as plsc

N = 320000
D = 128
S = 10000
NW = 32
SPW = 320
S_PAD = NW * SPW
C = 256
SUB = 8
BCH = SUB * C
RPW = SPW + 8
CNT_PAD = ((SPW + 15) // 16 + 1) * 16
NB = 2
CH = C // 2


def _pool_kernel(x_hbm, b_hbm, bounds_hbm, out_hbm,
                 xbuf, bbuf, idxb, cnt, bnds, acc_sh,
                 sx0, sx1, ss0, ss1):
    cid = lax.axis_index("c")
    sid = lax.axis_index("s")
    w = sid * 2 + cid
    seg_lo = w * SPW
    base = sid * RPW

    sx = (sx0, sx1)
    ss = (ss0, ss1)

    pltpu.sync_copy(bounds_hbm, bnds)
    bv0 = bnds[pl.ds(w, 16)]
    lo = bv0[0]
    hi = bv0[1]

    zeros16 = jnp.zeros((16,), jnp.float32)

    def zrow(i, carry):
        for j in range(8):
            xbuf[0, i, pl.ds(j * 16, 16)] = zeros16
        return carry
    lax.fori_loop(0, C, zrow, 0)
    pltpu.sync_copy(xbuf.at[0], acc_sh.at[pl.ds(base, C)])
    pltpu.sync_copy(xbuf.at[0].at[pl.ds(0, RPW - C)],
                    acc_sh.at[pl.ds(base + C, RPW - C)])
    for j in range(CNT_PAD // 16):
        cnt[pl.ds(j * 16, 16)] = zeros16

    lo_al = lo & jnp.int32(~7)
    nchunks = (hi - lo_al + C - 1) // C

    def xload(k, b):
        r_eff = pl.multiple_of(jnp.minimum(lo_al + k * C, N - C), 8)
        pltpu.async_copy(x_hbm.at[pl.ds(r_eff, C)], xbuf.at[b], sx[b])

    def xwait(b):
        pltpu.make_async_copy(x_hbm.at[pl.ds(0, C)], xbuf.at[b],
                              sx[b]).wait()

    def scat_half(b, h):
        return (xbuf.at[b].at[pl.ds(h * CH, CH)],
                acc_sh.at[idxb.at[b, h]], ss[b])

    def scat_wait(b):
        for h in range(2):
            src, dst, sem = scat_half(b, h)
            pltpu.make_async_copy(src, dst, sem).wait()

    @pl.when(nchunks > 0)
    def _():
        xload(0, 0)

    def pair(p, carry):
        for b in range(NB):
            k = NB * p + b

            @pl.when(k < nchunks)
            def _():
                s_sup = k // SUB
                rb_eff = pl.multiple_of(
                    jnp.minimum(lo_al + s_sup * BCH, N - BCH), 8)

                if b == 0:
                    @pl.when(lax.rem(k, SUB) == 0)
                    def _():
                        pltpu.sync_copy(b_hbm.at[pl.ds(rb_eff, BCH)], bbuf)

                @pl.when(k >= NB - 1)
                def _():
                    scat_wait((b + 1) % NB)

                @pl.when(k + 1 < nchunks)
                def _():
                    xload(k + 1, (b + 1) % NB)

                r = lo_al + k * C
                r_eff = pl.multiple_of(jnp.minimum(r, N - C), 8)
                off = r_eff - rb_eff
                vlo = jnp.maximum(r, lo)
                for j in range(C // 16):
                    bv = bbuf[pl.ds(off + j * 16, 16)]
                    g = r_eff + j * 16 + lax.iota(jnp.int32, 16)
                    valid = (g >= vlo) & (g < hi)
                    loc = jnp.where(valid, bv - seg_lo, SPW)
                    idxb[b, j // (CH // 16),
                         pl.ds((j % (CH // 16)) * 16, 16)] = base + loc
                    ones = jnp.where(valid, 1.0, 0.0).astype(jnp.float32)
                    plsc.addupdate_scatter(cnt, [loc], ones)

                xwait(b)
                for h in range(2):
                    src, dst, sem = scat_half(b, h)
                    pltpu.async_copy(src, dst, sem, add=True)
        return carry
    lax.fori_loop(0, (nchunks + NB - 1) // NB, pair, 0)

    for b in range(NB):
        cond = jnp.bool_(False)
        for t in range(1, NB):
            cond = cond | ((nchunks >= t) & (lax.rem(nchunks - t, NB) == b))

        @pl.when(cond)
        def _(b=b):
            scat_wait(b)

    W = 64
    blk = xbuf.at[0].at[pl.ds(0, W)]
    for t in range(SPW // W):
        pltpu.sync_copy(acc_sh.at[pl.ds(base + t * W, W)], blk)

        def div_row(s, carry, t=t):
            cv = cnt[pl.ds(t * W + s, 16)]
            inv = (jnp.ones((16,), jnp.float32) / jnp.maximum(cv, 1.0))[0]
            for j in range(8):
                xbuf[0, s, pl.ds(j * 16, 16)] = (
                    xbuf[0, s, pl.ds(j * 16, 16)] * inv)
            return carry
        lax.fori_loop(0, W, div_row, 0)
        pltpu.sync_copy(blk, out_hbm.at[pl.ds(seg_lo + t * W, W)])


def kernel(x, batch):
    b32 = batch.astype(jnp.int32)
    edges = jnp.arange(48, dtype=jnp.int32) * SPW
    bounds = jnp.sum((b32[:, None] < edges[None, :]).astype(jnp.int32),
                     axis=0, dtype=jnp.int32)

    mesh = plsc.VectorSubcoreMesh(core_axis_name="c", subcore_axis_name="s")
    out = pl.kernel(
        _pool_kernel,
        mesh=mesh,
        compiler_params=pltpu.CompilerParams(needs_layout_passes=False),
        out_type=jax.ShapeDtypeStruct((S_PAD, D), jnp.float32),
        scratch_types=[
            pltpu.VMEM((NB, C, D), jnp.float32),
            pltpu.VMEM((BCH,), jnp.int32),
            pltpu.VMEM((NB, 2, CH), jnp.int32),
            pltpu.VMEM((CNT_PAD,), jnp.float32),
            pltpu.VMEM((48,), jnp.int32),
            pltpu.VMEM_SHARED((16 * RPW, D), jnp.float32),
            pltpu.SemaphoreType.DMA,
            pltpu.SemaphoreType.DMA,
            pltpu.SemaphoreType.DMA,
            pltpu.SemaphoreType.DMA,
        ],
    )(x, b32, bounds)
    return out[:S]
